# baseline (device time: 56671 ns/iter reference)
import jax
import jax.numpy as jnp
from jax import lax
from jax.experimental import pallas as pl
from jax.experimental.pallas import tpu as pltpu

N_DEV = 4
WINDOW = 128
NEG = -1e9
NCHUNK = 2

_DevId = getattr(pl, "DeviceIdType", None) or pltpu.DeviceIdType
_sem_signal = getattr(pl, "semaphore_signal", None) or pltpu.semaphore_signal
_sem_wait = getattr(pl, "semaphore_wait", None) or pltpu.semaphore_wait


def kernel(x, Wq, K_ext, V_ext, Wo):
    B, Sq, D = x.shape
    Skv_sh = K_ext.shape[1]
    Dh = 64
    H = Wq.shape[1] // Dh

    def body(x_ref, wq_ref, k_ref, v_ref, wo_ref, out_ref,
             ksend, vsend, krecv, vrecv, pbuf, precv,
             ksend_sems, vsend_sems, krecv_sems, vrecv_sems,
             psend_sems, precv_sems):
        my = lax.axis_index("i")

        bar = pltpu.get_barrier_semaphore()
        for k in range(1, N_DEV):
            _sem_signal(bar, inc=1, device_id=((my + k) % N_DEV,),
                        device_id_type=_DevId.MESH)
        _sem_wait(bar, N_DEV - 1)

        for s in range(NCHUNK):
            @pl.when(my == s)
            def _(s=s):
                krecv[s] = k_ref[:, :, s * H:(s + 1) * H, :].astype(jnp.bfloat16)
                vrecv[s] = v_ref[:, :, s * H:(s + 1) * H, :].astype(jnp.bfloat16)
                cnt = 0
                for j in range(N_DEV):
                    if j == s:
                        continue
                    ksend[cnt] = k_ref[:, :, j * H:(j + 1) * H, :].astype(jnp.bfloat16)
                    vsend[cnt] = v_ref[:, :, j * H:(j + 1) * H, :].astype(jnp.bfloat16)
                    pltpu.make_async_remote_copy(
                        src_ref=ksend.at[cnt], dst_ref=krecv.at[s],
                        send_sem=ksend_sems.at[cnt], recv_sem=krecv_sems.at[s],
                        device_id=(j,), device_id_type=_DevId.MESH,
                    ).start()
                    pltpu.make_async_remote_copy(
                        src_ref=vsend.at[cnt], dst_ref=vrecv.at[s],
                        send_sem=vsend_sems.at[cnt], recv_sem=vrecv_sems.at[s],
                        device_id=(j,), device_id_type=_DevId.MESH,
                    ).start()
                    cnt += 1

        Q = []
        for b in range(B):
            Q.append(lax.dot_general(
                x_ref[b].astype(jnp.bfloat16), wq_ref[:, :].astype(jnp.bfloat16),
                (((1,), (0,)), ((), ())), preferred_element_type=jnp.float32))

        for c in range(NCHUNK):
            @pl.when(my != c)
            def _(c=c):
                pltpu.make_async_remote_copy(
                    src_ref=krecv.at[c], dst_ref=krecv.at[c],
                    send_sem=ksend_sems.at[0], recv_sem=krecv_sems.at[c],
                    device_id=(0,), device_id_type=_DevId.MESH,
                ).wait_recv()
                pltpu.make_async_remote_copy(
                    src_ref=vrecv.at[c], dst_ref=vrecv.at[c],
                    send_sem=vsend_sems.at[0], recv_sem=vrecv_sems.at[c],
                    device_id=(0,), device_id_type=_DevId.MESH,
                ).wait_recv()

        for s in range(NCHUNK):
            @pl.when(my == s)
            def _(s=s):
                for cnt in range(N_DEV - 1):
                    pltpu.make_async_remote_copy(
                        src_ref=ksend.at[cnt], dst_ref=krecv.at[s],
                        send_sem=ksend_sems.at[cnt], recv_sem=krecv_sems.at[s],
                        device_id=(0,), device_id_type=_DevId.MESH,
                    ).wait_send()
                    pltpu.make_async_remote_copy(
                        src_ref=vsend.at[cnt], dst_ref=vrecv.at[s],
                        send_sem=vsend_sems.at[cnt], recv_sem=vrecv_sems.at[s],
                        device_id=(0,), device_id_type=_DevId.MESH,
                    ).wait_send()

        Skv = NCHUNK * Skv_sh
        rows = lax.broadcasted_iota(jnp.int32, (Sq, Skv), 0)
        cols = lax.broadcasted_iota(jnp.int32, (Sq, Skv), 1)
        mask = jnp.abs(rows - cols) <= WINDOW

        for b in range(B):
            acc = jnp.zeros((Sq, D), jnp.float32)
            for h in range(H):
                q = Q[b][:, h * Dh:(h + 1) * Dh].astype(jnp.bfloat16)
                kcat = jnp.concatenate(
                    [krecv[c, b, :, h, :] for c in range(NCHUNK)], axis=0)
                s = lax.dot_general(
                    q, kcat, (((1,), (1,)), ((), ())),
                    preferred_element_type=jnp.float32) * 0.125
                s = jnp.where(mask, s, NEG)
                m = jnp.max(s, axis=1, keepdims=True)
                w = jnp.exp(s - m)
                w = w / jnp.sum(w, axis=1, keepdims=True)
                vcat = jnp.concatenate(
                    [vrecv[c, b, :, h, :] for c in range(NCHUNK)], axis=0)
                ctx = lax.dot_general(
                    w.astype(jnp.bfloat16), vcat, (((1,), (0,)), ((), ())),
                    preferred_element_type=jnp.float32)
                acc = acc + lax.dot_general(
                    ctx.astype(jnp.bfloat16),
                    wo_ref[h * Dh:(h + 1) * Dh, :].astype(jnp.bfloat16),
                    (((1,), (0,)), ((), ())), preferred_element_type=jnp.float32)
            pbuf[b] = acc.astype(jnp.bfloat16)
            out_ref[b] = acc

        for k in range(1, N_DEV):
            pltpu.make_async_remote_copy(
                src_ref=pbuf, dst_ref=precv.at[N_DEV - 1 - k],
                send_sem=psend_sems.at[k - 1],
                recv_sem=precv_sems.at[N_DEV - 1 - k],
                device_id=((my + k) % N_DEV,), device_id_type=_DevId.MESH,
            ).start()
        for j in range(N_DEV - 1):
            pltpu.make_async_remote_copy(
                src_ref=pbuf, dst_ref=precv.at[j],
                send_sem=psend_sems.at[0], recv_sem=precv_sems.at[j],
                device_id=(0,), device_id_type=_DevId.MESH,
            ).wait_recv()
        for b in range(B):
            tot = out_ref[b]
            for j in range(N_DEV - 1):
                tot = tot + precv[j, b].astype(jnp.float32)
            out_ref[b] = tot
        for k in range(1, N_DEV):
            pltpu.make_async_remote_copy(
                src_ref=pbuf, dst_ref=precv.at[0],
                send_sem=psend_sems.at[k - 1], recv_sem=precv_sems.at[0],
                device_id=(0,), device_id_type=_DevId.MESH,
            ).wait_send()

    return pl.pallas_call(
        body,
        out_shape=jax.ShapeDtypeStruct((B, Sq, D), jnp.float32),
        in_specs=[pl.BlockSpec(memory_space=pltpu.VMEM)] * 5,
        out_specs=pl.BlockSpec(memory_space=pltpu.VMEM),
        scratch_shapes=[
            pltpu.VMEM((N_DEV - 1, B, Skv_sh, H, Dh), jnp.bfloat16),
            pltpu.VMEM((N_DEV - 1, B, Skv_sh, H, Dh), jnp.bfloat16),
            pltpu.VMEM((NCHUNK, B, Skv_sh, H, Dh), jnp.bfloat16),
            pltpu.VMEM((NCHUNK, B, Skv_sh, H, Dh), jnp.bfloat16),
            pltpu.VMEM((B, Sq, D), jnp.bfloat16),
            pltpu.VMEM((N_DEV - 1, B, Sq, D), jnp.bfloat16),
            pltpu.SemaphoreType.DMA((N_DEV - 1,)),
            pltpu.SemaphoreType.DMA((N_DEV - 1,)),
            pltpu.SemaphoreType.DMA((NCHUNK,)),
            pltpu.SemaphoreType.DMA((NCHUNK,)),
            pltpu.SemaphoreType.DMA((N_DEV - 1,)),
            pltpu.SemaphoreType.DMA((N_DEV - 1,)),
        ],
        compiler_params=pltpu.CompilerParams(collective_id=0),
    )(x, Wq, K_ext, V_ext, Wo)


# device time: 37190 ns/iter; 1.5238x vs baseline; 1.5238x over previous
import jax
import jax.numpy as jnp
from jax import lax
from jax.experimental import pallas as pl
from jax.experimental.pallas import tpu as pltpu

N_DEV = 4
WINDOW = 128
NEG = -1e9
CHUNK_ROWS = (256, 128)
NCHUNK = len(CHUNK_ROWS)

_DevId = getattr(pl, "DeviceIdType", None) or pltpu.DeviceIdType
_sem_signal = getattr(pl, "semaphore_signal", None) or pltpu.semaphore_signal
_sem_wait = getattr(pl, "semaphore_wait", None) or pltpu.semaphore_wait


def kernel(x, Wq, K_ext, V_ext, Wo):
    B, Sq, D = x.shape
    Skv_sh = K_ext.shape[1]
    Dh = 64
    H = Wq.shape[1] // Dh
    HD = H * Dh
    Skv = sum(CHUNK_ROWS)

    K2 = K_ext.reshape(B, Skv_sh, -1)
    V2 = V_ext.reshape(B, Skv_sh, -1)

    def body(x_ref, wq_ref, k_ref, v_ref, wo_ref, out_ref,
             ksend0, vsend0, ksend1, vsend1, krecv0, vrecv0, krecv1, vrecv1,
             pbuf, precv,
             ksend_sems, vsend_sems, krecv_sems, vrecv_sems,
             psend_sems, precv_sems):
        my = lax.axis_index("i")

        bar = pltpu.get_barrier_semaphore()
        for k in range(1, N_DEV):
            _sem_signal(bar, inc=1, device_id=((my + k) % N_DEV,),
                        device_id_type=_DevId.MESH)
        _sem_wait(bar, N_DEV - 1)

        for s, (ks, vs, kr, vr, rows) in enumerate(
                ((ksend0, vsend0, krecv0, vrecv0, CHUNK_ROWS[0]),
                 (ksend1, vsend1, krecv1, vrecv1, CHUNK_ROWS[1]))):
            @pl.when(my == s)
            def _(s=s, ks=ks, vs=vs, kr=kr, vr=vr, rows=rows):
                kr[:, :, :] = k_ref[:, :rows, s * HD:(s + 1) * HD].astype(jnp.bfloat16)
                vr[:, :, :] = v_ref[:, :rows, s * HD:(s + 1) * HD].astype(jnp.bfloat16)
                dsts = [(s + 2) % N_DEV, (s + 1) % N_DEV, (s + 3) % N_DEV]
                for cnt, j in enumerate(dsts):
                    ks[cnt] = k_ref[:, :rows, j * HD:(j + 1) * HD].astype(jnp.bfloat16)
                    vs[cnt] = v_ref[:, :rows, j * HD:(j + 1) * HD].astype(jnp.bfloat16)
                    pltpu.make_async_remote_copy(
                        src_ref=ks.at[cnt], dst_ref=kr,
                        send_sem=ksend_sems.at[cnt], recv_sem=krecv_sems.at[s],
                        device_id=(j,), device_id_type=_DevId.MESH,
                    ).start()
                    pltpu.make_async_remote_copy(
                        src_ref=vs.at[cnt], dst_ref=vr,
                        send_sem=vsend_sems.at[cnt], recv_sem=vrecv_sems.at[s],
                        device_id=(j,), device_id_type=_DevId.MESH,
                    ).start()

        Q = []
        for b in range(B):
            Q.append(lax.dot_general(
                x_ref[b].astype(jnp.bfloat16), wq_ref[:, :].astype(jnp.bfloat16),
                (((1,), (0,)), ((), ())), preferred_element_type=jnp.float32))

        for c, (kr, vr) in enumerate(((krecv0, vrecv0), (krecv1, vrecv1))):
            @pl.when(my != c)
            def _(c=c, kr=kr, vr=vr):
                pltpu.make_async_remote_copy(
                    src_ref=kr, dst_ref=kr,
                    send_sem=ksend_sems.at[0], recv_sem=krecv_sems.at[c],
                    device_id=(0,), device_id_type=_DevId.MESH,
                ).wait_recv()
                pltpu.make_async_remote_copy(
                    src_ref=vr, dst_ref=vr,
                    send_sem=vsend_sems.at[0], recv_sem=vrecv_sems.at[c],
                    device_id=(0,), device_id_type=_DevId.MESH,
                ).wait_recv()

        for s, (ks, vs, kr, vr) in enumerate(
                ((ksend0, vsend0, krecv0, vrecv0),
                 (ksend1, vsend1, krecv1, vrecv1))):
            @pl.when(my == s)
            def _(s=s, ks=ks, vs=vs, kr=kr, vr=vr):
                for cnt in range(N_DEV - 1):
                    pltpu.make_async_remote_copy(
                        src_ref=ks.at[cnt], dst_ref=kr,
                        send_sem=ksend_sems.at[cnt], recv_sem=krecv_sems.at[s],
                        device_id=(0,), device_id_type=_DevId.MESH,
                    ).wait_send()
                    pltpu.make_async_remote_copy(
                        src_ref=vs.at[cnt], dst_ref=vr,
                        send_sem=vsend_sems.at[cnt], recv_sem=vrecv_sems.at[s],
                        device_id=(0,), device_id_type=_DevId.MESH,
                    ).wait_send()

        rows_i = lax.broadcasted_iota(jnp.int32, (Sq, Skv), 0)
        cols_i = lax.broadcasted_iota(jnp.int32, (Sq, Skv), 1)
        mask = jnp.abs(rows_i - cols_i) <= WINDOW

        accs = []
        for b in range(B):
            acc = jnp.zeros((Sq, D), jnp.float32)
            for h in range(H):
                q = Q[b][:, h * Dh:(h + 1) * Dh].astype(jnp.bfloat16)
                kcat = jnp.concatenate(
                    [krecv0[b, :, h * Dh:(h + 1) * Dh],
                     krecv1[b, :, h * Dh:(h + 1) * Dh]], axis=0)
                s = lax.dot_general(
                    q, kcat, (((1,), (1,)), ((), ())),
                    preferred_element_type=jnp.float32) * 0.125
                s = jnp.where(mask, s, NEG)
                m = jnp.max(s, axis=1, keepdims=True)
                w = jnp.exp(s - m)
                w = w / jnp.sum(w, axis=1, keepdims=True)
                vcat = jnp.concatenate(
                    [vrecv0[b, :, h * Dh:(h + 1) * Dh],
                     vrecv1[b, :, h * Dh:(h + 1) * Dh]], axis=0)
                ctx = lax.dot_general(
                    w.astype(jnp.bfloat16), vcat, (((1,), (0,)), ((), ())),
                    preferred_element_type=jnp.float32)
                acc = acc + lax.dot_general(
                    ctx.astype(jnp.bfloat16),
                    wo_ref[h * Dh:(h + 1) * Dh, :].astype(jnp.bfloat16),
                    (((1,), (0,)), ((), ())), preferred_element_type=jnp.float32)
            accs.append(acc)
            pbuf[b] = acc.astype(jnp.bfloat16)
            for k in (2, 1, 3):
                pltpu.make_async_remote_copy(
                    src_ref=pbuf.at[b], dst_ref=precv.at[N_DEV - 1 - k, b],
                    send_sem=psend_sems.at[k - 1, b],
                    recv_sem=precv_sems.at[N_DEV - 1 - k, b],
                    device_id=((my + k) % N_DEV,), device_id_type=_DevId.MESH,
                ).start()

        for b in range(B):
            out_ref[b] = accs[b]

        for j in range(N_DEV - 1):
            for b in range(B):
                pltpu.make_async_remote_copy(
                    src_ref=pbuf.at[b], dst_ref=precv.at[j, b],
                    send_sem=psend_sems.at[0, b], recv_sem=precv_sems.at[j, b],
                    device_id=(0,), device_id_type=_DevId.MESH,
                ).wait_recv()
        for b in range(B):
            tot = out_ref[b]
            for j in range(N_DEV - 1):
                tot = tot + precv[j, b].astype(jnp.float32)
            out_ref[b] = tot
        for k in range(1, N_DEV):
            for b in range(B):
                pltpu.make_async_remote_copy(
                    src_ref=pbuf.at[b], dst_ref=precv.at[0, b],
                    send_sem=psend_sems.at[k - 1, b], recv_sem=precv_sems.at[0, b],
                    device_id=(0,), device_id_type=_DevId.MESH,
                ).wait_send()

    return pl.pallas_call(
        body,
        out_shape=jax.ShapeDtypeStruct((B, Sq, D), jnp.float32),
        in_specs=[pl.BlockSpec(memory_space=pltpu.VMEM)] * 5,
        out_specs=pl.BlockSpec(memory_space=pltpu.VMEM),
        scratch_shapes=[
            pltpu.VMEM((N_DEV - 1, B, CHUNK_ROWS[0], HD), jnp.bfloat16),
            pltpu.VMEM((N_DEV - 1, B, CHUNK_ROWS[0], HD), jnp.bfloat16),
            pltpu.VMEM((N_DEV - 1, B, CHUNK_ROWS[1], HD), jnp.bfloat16),
            pltpu.VMEM((N_DEV - 1, B, CHUNK_ROWS[1], HD), jnp.bfloat16),
            pltpu.VMEM((B, CHUNK_ROWS[0], HD), jnp.bfloat16),
            pltpu.VMEM((B, CHUNK_ROWS[0], HD), jnp.bfloat16),
            pltpu.VMEM((B, CHUNK_ROWS[1], HD), jnp.bfloat16),
            pltpu.VMEM((B, CHUNK_ROWS[1], HD), jnp.bfloat16),
            pltpu.VMEM((B, Sq, D), jnp.bfloat16),
            pltpu.VMEM((N_DEV - 1, B, Sq, D), jnp.bfloat16),
            pltpu.SemaphoreType.DMA((N_DEV - 1,)),
            pltpu.SemaphoreType.DMA((N_DEV - 1,)),
            pltpu.SemaphoreType.DMA((NCHUNK,)),
            pltpu.SemaphoreType.DMA((NCHUNK,)),
            pltpu.SemaphoreType.DMA((N_DEV - 1, B)),
            pltpu.SemaphoreType.DMA((N_DEV - 1, B)),
        ],
        compiler_params=pltpu.CompilerParams(collective_id=0),
    )(x, Wq, K2, V2, Wo)


# device time: 32356 ns/iter; 1.7515x vs baseline; 1.1494x over previous
import jax
import jax.numpy as jnp
from jax import lax
from jax.experimental import pallas as pl
from jax.experimental.pallas import tpu as pltpu

N_DEV = 4
WINDOW = 128
NEG = -1e9
CHUNK_ROWS = (256, 128)
NCHUNK = len(CHUNK_ROWS)

_DevId = getattr(pl, "DeviceIdType", None) or pltpu.DeviceIdType
_sem_signal = getattr(pl, "semaphore_signal", None) or pltpu.semaphore_signal
_sem_wait = getattr(pl, "semaphore_wait", None) or pltpu.semaphore_wait


def kernel(x, Wq, K_ext, V_ext, Wo):
    B, Sq, D = x.shape
    Dh = 64
    H = Wq.shape[1] // Dh
    HD = H * Dh
    Skv = sum(CHUNK_ROWS)
    SqQ = Sq // N_DEV

    K2 = K_ext.reshape(B, K_ext.shape[1], -1)
    V2 = V_ext.reshape(B, V_ext.shape[1], -1)

    def body(x_ref, wq_ref, k_ref, v_ref, wo_ref, out_ref,
             ksend0, vsend0, ksend1, vsend1, krecv0, vrecv0, krecv1, vrecv1,
             pbuf, rsrecv, agsend, agrecv,
             ksend_sems, vsend_sems, krecv_sems, vrecv_sems,
             rssend_sems, rsrecv_sems, agsend_sems, agrecv_sems):
        my = lax.axis_index("i")

        bar = pltpu.get_barrier_semaphore()
        for k in range(1, N_DEV):
            _sem_signal(bar, inc=1, device_id=((my + k) % N_DEV,),
                        device_id_type=_DevId.MESH)
        _sem_wait(bar, N_DEV - 1)

        for s, (ks, vs, kr, vr, rows) in enumerate(
                ((ksend0, vsend0, krecv0, vrecv0, CHUNK_ROWS[0]),
                 (ksend1, vsend1, krecv1, vrecv1, CHUNK_ROWS[1]))):
            @pl.when(my == s)
            def _(s=s, ks=ks, vs=vs, kr=kr, vr=vr, rows=rows):
                dsts = [(s + 2) % N_DEV, (s + 1) % N_DEV, (s + 3) % N_DEV]
                for cnt, j in enumerate(dsts):
                    ks[cnt] = k_ref[:, :rows, j * HD:(j + 1) * HD].astype(jnp.bfloat16)
                    pltpu.make_async_remote_copy(
                        src_ref=ks.at[cnt], dst_ref=kr,
                        send_sem=ksend_sems.at[cnt], recv_sem=krecv_sems.at[s],
                        device_id=(j,), device_id_type=_DevId.MESH,
                    ).start()
                for cnt, j in enumerate(dsts):
                    vs[cnt] = v_ref[:, :rows, j * HD:(j + 1) * HD].astype(jnp.bfloat16)
                    pltpu.make_async_remote_copy(
                        src_ref=vs.at[cnt], dst_ref=vr,
                        send_sem=vsend_sems.at[cnt], recv_sem=vrecv_sems.at[s],
                        device_id=(j,), device_id_type=_DevId.MESH,
                    ).start()
                kr[:, :, :] = k_ref[:, :rows, s * HD:(s + 1) * HD].astype(jnp.bfloat16)
                vr[:, :, :] = v_ref[:, :rows, s * HD:(s + 1) * HD].astype(jnp.bfloat16)

        Q = []
        for b in range(B):
            Q.append(lax.dot_general(
                x_ref[b].astype(jnp.bfloat16), wq_ref[:, :].astype(jnp.bfloat16),
                (((1,), (0,)), ((), ())), preferred_element_type=jnp.float32))

        for c, (kr, vr) in enumerate(((krecv0, vrecv0), (krecv1, vrecv1))):
            @pl.when(my != c)
            def _(c=c, kr=kr, vr=vr):
                pltpu.make_async_remote_copy(
                    src_ref=kr, dst_ref=kr,
                    send_sem=ksend_sems.at[0], recv_sem=krecv_sems.at[c],
                    device_id=(0,), device_id_type=_DevId.MESH,
                ).wait_recv()
                pltpu.make_async_remote_copy(
                    src_ref=vr, dst_ref=vr,
                    send_sem=vsend_sems.at[0], recv_sem=vrecv_sems.at[c],
                    device_id=(0,), device_id_type=_DevId.MESH,
                ).wait_recv()

        for s, (ks, vs, kr, vr) in enumerate(
                ((ksend0, vsend0, krecv0, vrecv0),
                 (ksend1, vsend1, krecv1, vrecv1))):
            @pl.when(my == s)
            def _(s=s, ks=ks, vs=vs, kr=kr, vr=vr):
                for cnt in range(N_DEV - 1):
                    pltpu.make_async_remote_copy(
                        src_ref=ks.at[cnt], dst_ref=kr,
                        send_sem=ksend_sems.at[cnt], recv_sem=krecv_sems.at[s],
                        device_id=(0,), device_id_type=_DevId.MESH,
                    ).wait_send()
                    pltpu.make_async_remote_copy(
                        src_ref=vs.at[cnt], dst_ref=vr,
                        send_sem=vsend_sems.at[cnt], recv_sem=vrecv_sems.at[s],
                        device_id=(0,), device_id_type=_DevId.MESH,
                    ).wait_send()

        rows_i = lax.broadcasted_iota(jnp.int32, (Sq, Skv), 0)
        cols_i = lax.broadcasted_iota(jnp.int32, (Sq, Skv), 1)
        mask = jnp.abs(rows_i - cols_i) <= WINDOW

        accs = []
        for b in range(B):
            acc = jnp.zeros((Sq, D), jnp.float32)
            for h in range(H):
                q = Q[b][:, h * Dh:(h + 1) * Dh].astype(jnp.bfloat16)
                kcat = jnp.concatenate(
                    [krecv0[b, :, h * Dh:(h + 1) * Dh],
                     krecv1[b, :, h * Dh:(h + 1) * Dh]], axis=0)
                s = lax.dot_general(
                    q, kcat, (((1,), (1,)), ((), ())),
                    preferred_element_type=jnp.float32) * 0.125
                s = jnp.where(mask, s, NEG)
                m = jnp.max(s, axis=1, keepdims=True)
                w = jnp.exp(s - m)
                w = w / jnp.sum(w, axis=1, keepdims=True)
                vcat = jnp.concatenate(
                    [vrecv0[b, :, h * Dh:(h + 1) * Dh],
                     vrecv1[b, :, h * Dh:(h + 1) * Dh]], axis=0)
                ctx = lax.dot_general(
                    w.astype(jnp.bfloat16), vcat, (((1,), (0,)), ((), ())),
                    preferred_element_type=jnp.float32)
                acc = acc + lax.dot_general(
                    ctx.astype(jnp.bfloat16),
                    wo_ref[h * Dh:(h + 1) * Dh, :].astype(jnp.bfloat16),
                    (((1,), (0,)), ((), ())), preferred_element_type=jnp.float32)
            accs.append(acc)
            pbuf[b] = acc.astype(jnp.bfloat16)
            for k in (2, 1, 3):
                d = (my + k) % N_DEV
                pltpu.make_async_remote_copy(
                    src_ref=pbuf.at[b, pl.ds(d * SqQ, SqQ), :],
                    dst_ref=rsrecv.at[N_DEV - 1 - k, b],
                    send_sem=rssend_sems.at[k - 1, b],
                    recv_sem=rsrecv_sems.at[N_DEV - 1 - k, b],
                    device_id=(d,), device_id_type=_DevId.MESH,
                ).start()

        sums = []
        for b in range(B):
            for j in range(N_DEV - 1):
                pltpu.make_async_remote_copy(
                    src_ref=rsrecv.at[j, b], dst_ref=rsrecv.at[j, b],
                    send_sem=rssend_sems.at[0, b], recv_sem=rsrecv_sems.at[j, b],
                    device_id=(0,), device_id_type=_DevId.MESH,
                ).wait_recv()
            sum_q = pbuf[b, pl.ds(my * SqQ, SqQ), :].astype(jnp.float32)
            for j in range(N_DEV - 1):
                sum_q = sum_q + rsrecv[j, b].astype(jnp.float32)
            sums.append(sum_q)
            agsend[b] = sum_q.astype(jnp.bfloat16)
            for k in (2, 1, 3):
                pltpu.make_async_remote_copy(
                    src_ref=agsend.at[b], dst_ref=agrecv.at[N_DEV - 1 - k, b],
                    send_sem=agsend_sems.at[k - 1, b],
                    recv_sem=agrecv_sems.at[N_DEV - 1 - k, b],
                    device_id=((my + k) % N_DEV,), device_id_type=_DevId.MESH,
                ).start()

        for b in range(B):
            out_ref[b, pl.ds(my * SqQ, SqQ), :] = sums[b]
            for j in range(N_DEV - 1):
                pltpu.make_async_remote_copy(
                    src_ref=agsend.at[b], dst_ref=agrecv.at[j, b],
                    send_sem=agsend_sems.at[0, b], recv_sem=agrecv_sems.at[j, b],
                    device_id=(0,), device_id_type=_DevId.MESH,
                ).wait_recv()
                src = (my + j + 1) % N_DEV
                out_ref[b, pl.ds(src * SqQ, SqQ), :] = \
                    agrecv[j, b].astype(jnp.float32)

        for b in range(B):
            for k in range(1, N_DEV):
                pltpu.make_async_remote_copy(
                    src_ref=pbuf.at[b, pl.ds(0, SqQ), :], dst_ref=rsrecv.at[0, b],
                    send_sem=rssend_sems.at[k - 1, b], recv_sem=rsrecv_sems.at[0, b],
                    device_id=(0,), device_id_type=_DevId.MESH,
                ).wait_send()
                pltpu.make_async_remote_copy(
                    src_ref=agsend.at[b], dst_ref=agrecv.at[0, b],
                    send_sem=agsend_sems.at[k - 1, b], recv_sem=agrecv_sems.at[0, b],
                    device_id=(0,), device_id_type=_DevId.MESH,
                ).wait_send()

    return pl.pallas_call(
        body,
        out_shape=jax.ShapeDtypeStruct((B, Sq, D), jnp.float32),
        in_specs=[pl.BlockSpec(memory_space=pltpu.VMEM)] * 5,
        out_specs=pl.BlockSpec(memory_space=pltpu.VMEM),
        scratch_shapes=[
            pltpu.VMEM((N_DEV - 1, B, CHUNK_ROWS[0], HD), jnp.bfloat16),
            pltpu.VMEM((N_DEV - 1, B, CHUNK_ROWS[0], HD), jnp.bfloat16),
            pltpu.VMEM((N_DEV - 1, B, CHUNK_ROWS[1], HD), jnp.bfloat16),
            pltpu.VMEM((N_DEV - 1, B, CHUNK_ROWS[1], HD), jnp.bfloat16),
            pltpu.VMEM((B, CHUNK_ROWS[0], HD), jnp.bfloat16),
            pltpu.VMEM((B, CHUNK_ROWS[0], HD), jnp.bfloat16),
            pltpu.VMEM((B, CHUNK_ROWS[1], HD), jnp.bfloat16),
            pltpu.VMEM((B, CHUNK_ROWS[1], HD), jnp.bfloat16),
            pltpu.VMEM((B, Sq, D), jnp.bfloat16),
            pltpu.VMEM((N_DEV - 1, B, Sq // N_DEV, D), jnp.bfloat16),
            pltpu.VMEM((B, Sq // N_DEV, D), jnp.bfloat16),
            pltpu.VMEM((N_DEV - 1, B, Sq // N_DEV, D), jnp.bfloat16),
            pltpu.SemaphoreType.DMA((N_DEV - 1,)),
            pltpu.SemaphoreType.DMA((N_DEV - 1,)),
            pltpu.SemaphoreType.DMA((NCHUNK,)),
            pltpu.SemaphoreType.DMA((NCHUNK,)),
            pltpu.SemaphoreType.DMA((N_DEV - 1, B)),
            pltpu.SemaphoreType.DMA((N_DEV - 1, B)),
            pltpu.SemaphoreType.DMA((N_DEV - 1, B)),
            pltpu.SemaphoreType.DMA((N_DEV - 1, B)),
        ],
        compiler_params=pltpu.CompilerParams(collective_id=0),
    )(x, Wq, K2, V2, Wo)


# device time: 31208 ns/iter; 1.8159x vs baseline; 1.0368x over previous
import jax
import jax.numpy as jnp
from jax import lax
from jax.experimental import pallas as pl
from jax.experimental.pallas import tpu as pltpu

N_DEV = 4
WINDOW = 128
NEG = -1e9
CHUNK_ROWS = (256, 128)
NCHUNK = len(CHUNK_ROWS)

_DevId = getattr(pl, "DeviceIdType", None) or pltpu.DeviceIdType
_sem_signal = getattr(pl, "semaphore_signal", None) or pltpu.semaphore_signal
_sem_wait = getattr(pl, "semaphore_wait", None) or pltpu.semaphore_wait


def kernel(x, Wq, K_ext, V_ext, Wo):
    B, Sq, D = x.shape
    Dh = 64
    H = Wq.shape[1] // Dh
    HD = H * Dh
    SqQ = Sq // N_DEV

    K2 = K_ext.reshape(B, K_ext.shape[1], -1)
    V2 = V_ext.reshape(B, V_ext.shape[1], -1)

    def body(x_ref, wq_ref, k_ref, v_ref, wo_ref, out_ref,
             ksend0, vsend0, ksend1, vsend1, krecv0, vrecv0, krecv1, vrecv1,
             pbuf, rsrecv, agsend, agrecv,
             ksend_sems, vsend_sems, krecv_sems, vrecv_sems,
             rssend_sems, rsrecv_sems, agsend_sems, agrecv_sems):
        my = lax.axis_index("i")

        bar = pltpu.get_barrier_semaphore()
        for k in range(1, N_DEV):
            _sem_signal(bar, inc=1, device_id=((my + k) % N_DEV,),
                        device_id_type=_DevId.MESH)
        _sem_wait(bar, N_DEV - 1)

        for s, (ks, vs, kr, vr, rows) in enumerate(
                ((ksend0, vsend0, krecv0, vrecv0, CHUNK_ROWS[0]),
                 (ksend1, vsend1, krecv1, vrecv1, CHUNK_ROWS[1]))):
            @pl.when(my == s)
            def _(s=s, ks=ks, vs=vs, kr=kr, vr=vr, rows=rows):
                dsts = [(s + 2) % N_DEV, (s + 1) % N_DEV, (s + 3) % N_DEV]
                for cnt, j in enumerate(dsts):
                    ks[cnt] = k_ref[:, :rows, j * HD:(j + 1) * HD].astype(jnp.bfloat16)
                    pltpu.make_async_remote_copy(
                        src_ref=ks.at[cnt], dst_ref=kr,
                        send_sem=ksend_sems.at[cnt], recv_sem=krecv_sems.at[s],
                        device_id=(j,), device_id_type=_DevId.MESH,
                    ).start()
                for cnt, j in enumerate(dsts):
                    vs[cnt] = v_ref[:, :rows, j * HD:(j + 1) * HD].astype(jnp.bfloat16)
                    pltpu.make_async_remote_copy(
                        src_ref=vs.at[cnt], dst_ref=vr,
                        send_sem=vsend_sems.at[cnt], recv_sem=vrecv_sems.at[s],
                        device_id=(j,), device_id_type=_DevId.MESH,
                    ).start()
                kr[:, :, :] = k_ref[:, :rows, s * HD:(s + 1) * HD].astype(jnp.bfloat16)
                vr[:, :, :] = v_ref[:, :rows, s * HD:(s + 1) * HD].astype(jnp.bfloat16)

        Q = []
        for b in range(B):
            Q.append((lax.dot_general(
                x_ref[b].astype(jnp.bfloat16), wq_ref[:, :].astype(jnp.bfloat16),
                (((1,), (0,)), ((), ())), preferred_element_type=jnp.float32)
                * 0.125).astype(jnp.bfloat16))

        def bias(rows, col0):
            r = lax.broadcasted_iota(jnp.int32, (Sq, rows), 0)
            c = lax.broadcasted_iota(jnp.int32, (Sq, rows), 1) + col0
            return jnp.where(jnp.abs(r - c) <= WINDOW, 0.0, NEG).astype(jnp.float32)

        bias0 = bias(CHUNK_ROWS[0], 0)
        bias1 = bias(CHUNK_ROWS[1], CHUNK_ROWS[0])

        def wait_chunk(c, kr, vr):
            @pl.when(my != c)
            def _():
                pltpu.make_async_remote_copy(
                    src_ref=kr, dst_ref=kr,
                    send_sem=ksend_sems.at[0], recv_sem=krecv_sems.at[c],
                    device_id=(0,), device_id_type=_DevId.MESH,
                ).wait_recv()
                pltpu.make_async_remote_copy(
                    src_ref=vr, dst_ref=vr,
                    send_sem=vsend_sems.at[0], recv_sem=vrecv_sems.at[c],
                    device_id=(0,), device_id_type=_DevId.MESH,
                ).wait_recv()

        wait_chunk(0, krecv0, vrecv0)
        c0 = [[None] * H for _ in range(B)]
        l0 = [[None] * H for _ in range(B)]
        for b in range(B):
            for h in range(H):
                q = Q[b][:, h * Dh:(h + 1) * Dh]
                e0 = jnp.exp(lax.dot_general(
                    q, krecv0[b, :, h * Dh:(h + 1) * Dh], (((1,), (1,)), ((), ())),
                    preferred_element_type=jnp.float32) + bias0)
                l0[b][h] = jnp.sum(e0, axis=1, keepdims=True)
                c0[b][h] = lax.dot_general(
                    e0.astype(jnp.bfloat16), vrecv0[b, :, h * Dh:(h + 1) * Dh],
                    (((1,), (0,)), ((), ())), preferred_element_type=jnp.float32)

        wait_chunk(1, krecv1, vrecv1)
        for b in range(B):
            acc = jnp.zeros((Sq, D), jnp.float32)
            for h in range(H):
                q = Q[b][:, h * Dh:(h + 1) * Dh]
                e1 = jnp.exp(lax.dot_general(
                    q, krecv1[b, :, h * Dh:(h + 1) * Dh], (((1,), (1,)), ((), ())),
                    preferred_element_type=jnp.float32) + bias1)
                ctx = c0[b][h] + lax.dot_general(
                    e1.astype(jnp.bfloat16), vrecv1[b, :, h * Dh:(h + 1) * Dh],
                    (((1,), (0,)), ((), ())), preferred_element_type=jnp.float32)
                ctx = ctx * (1.0 / (l0[b][h] + jnp.sum(e1, axis=1, keepdims=True)))
                acc = acc + lax.dot_general(
                    ctx.astype(jnp.bfloat16),
                    wo_ref[h * Dh:(h + 1) * Dh, :].astype(jnp.bfloat16),
                    (((1,), (0,)), ((), ())), preferred_element_type=jnp.float32)
            pbuf[b] = acc.astype(jnp.bfloat16)
            for k in (2, 1, 3):
                d = (my + k) % N_DEV
                pltpu.make_async_remote_copy(
                    src_ref=pbuf.at[b, pl.ds(d * SqQ, SqQ), :],
                    dst_ref=rsrecv.at[N_DEV - 1 - k, b],
                    send_sem=rssend_sems.at[k - 1, b],
                    recv_sem=rsrecv_sems.at[N_DEV - 1 - k, b],
                    device_id=(d,), device_id_type=_DevId.MESH,
                ).start()

        for s, (ks, vs, kr, vr) in enumerate(
                ((ksend0, vsend0, krecv0, vrecv0),
                 (ksend1, vsend1, krecv1, vrecv1))):
            @pl.when(my == s)
            def _(s=s, ks=ks, vs=vs, kr=kr, vr=vr):
                for cnt in range(N_DEV - 1):
                    pltpu.make_async_remote_copy(
                        src_ref=ks.at[cnt], dst_ref=kr,
                        send_sem=ksend_sems.at[cnt], recv_sem=krecv_sems.at[s],
                        device_id=(0,), device_id_type=_DevId.MESH,
                    ).wait_send()
                    pltpu.make_async_remote_copy(
                        src_ref=vs.at[cnt], dst_ref=vr,
                        send_sem=vsend_sems.at[cnt], recv_sem=vrecv_sems.at[s],
                        device_id=(0,), device_id_type=_DevId.MESH,
                    ).wait_send()

        sums = []
        for b in range(B):
            for j in range(N_DEV - 1):
                pltpu.make_async_remote_copy(
                    src_ref=rsrecv.at[j, b], dst_ref=rsrecv.at[j, b],
                    send_sem=rssend_sems.at[0, b], recv_sem=rsrecv_sems.at[j, b],
                    device_id=(0,), device_id_type=_DevId.MESH,
                ).wait_recv()
            sum_q = pbuf[b, pl.ds(my * SqQ, SqQ), :].astype(jnp.float32)
            for j in range(N_DEV - 1):
                sum_q = sum_q + rsrecv[j, b].astype(jnp.float32)
            sums.append(sum_q)
            agsend[b] = sum_q.astype(jnp.bfloat16)
            for k in (2, 1, 3):
                pltpu.make_async_remote_copy(
                    src_ref=agsend.at[b], dst_ref=agrecv.at[N_DEV - 1 - k, b],
                    send_sem=agsend_sems.at[k - 1, b],
                    recv_sem=agrecv_sems.at[N_DEV - 1 - k, b],
                    device_id=((my + k) % N_DEV,), device_id_type=_DevId.MESH,
                ).start()

        for b in range(B):
            out_ref[b, pl.ds(my * SqQ, SqQ), :] = sums[b]
            for j in range(N_DEV - 1):
                pltpu.make_async_remote_copy(
                    src_ref=agsend.at[b], dst_ref=agrecv.at[j, b],
                    send_sem=agsend_sems.at[0, b], recv_sem=agrecv_sems.at[j, b],
                    device_id=(0,), device_id_type=_DevId.MESH,
                ).wait_recv()
                src = (my + j + 1) % N_DEV
                out_ref[b, pl.ds(src * SqQ, SqQ), :] = \
                    agrecv[j, b].astype(jnp.float32)

        for b in range(B):
            for k in range(1, N_DEV):
                pltpu.make_async_remote_copy(
                    src_ref=pbuf.at[b, pl.ds(0, SqQ), :], dst_ref=rsrecv.at[0, b],
                    send_sem=rssend_sems.at[k - 1, b], recv_sem=rsrecv_sems.at[0, b],
                    device_id=(0,), device_id_type=_DevId.MESH,
                ).wait_send()
                pltpu.make_async_remote_copy(
                    src_ref=agsend.at[b], dst_ref=agrecv.at[0, b],
                    send_sem=agsend_sems.at[k - 1, b], recv_sem=agrecv_sems.at[0, b],
                    device_id=(0,), device_id_type=_DevId.MESH,
                ).wait_send()

    return pl.pallas_call(
        body,
        out_shape=jax.ShapeDtypeStruct((B, Sq, D), jnp.float32),
        in_specs=[pl.BlockSpec(memory_space=pltpu.VMEM)] * 5,
        out_specs=pl.BlockSpec(memory_space=pltpu.VMEM),
        scratch_shapes=[
            pltpu.VMEM((N_DEV - 1, B, CHUNK_ROWS[0], HD), jnp.bfloat16),
            pltpu.VMEM((N_DEV - 1, B, CHUNK_ROWS[0], HD), jnp.bfloat16),
            pltpu.VMEM((N_DEV - 1, B, CHUNK_ROWS[1], HD), jnp.bfloat16),
            pltpu.VMEM((N_DEV - 1, B, CHUNK_ROWS[1], HD), jnp.bfloat16),
            pltpu.VMEM((B, CHUNK_ROWS[0], HD), jnp.bfloat16),
            pltpu.VMEM((B, CHUNK_ROWS[0], HD), jnp.bfloat16),
            pltpu.VMEM((B, CHUNK_ROWS[1], HD), jnp.bfloat16),
            pltpu.VMEM((B, CHUNK_ROWS[1], HD), jnp.bfloat16),
            pltpu.VMEM((B, Sq, D), jnp.bfloat16),
            pltpu.VMEM((N_DEV - 1, B, Sq // N_DEV, D), jnp.bfloat16),
            pltpu.VMEM((B, Sq // N_DEV, D), jnp.bfloat16),
            pltpu.VMEM((N_DEV - 1, B, Sq // N_DEV, D), jnp.bfloat16),
            pltpu.SemaphoreType.DMA((N_DEV - 1,)),
            pltpu.SemaphoreType.DMA((N_DEV - 1,)),
            pltpu.SemaphoreType.DMA((NCHUNK,)),
            pltpu.SemaphoreType.DMA((NCHUNK,)),
            pltpu.SemaphoreType.DMA((N_DEV - 1, B)),
            pltpu.SemaphoreType.DMA((N_DEV - 1, B)),
            pltpu.SemaphoreType.DMA((N_DEV - 1, B)),
            pltpu.SemaphoreType.DMA((N_DEV - 1, B)),
        ],
        compiler_params=pltpu.CompilerParams(collective_id=0),
    )(x, Wq, K2, V2, Wo)


# device time: 30854 ns/iter; 1.8367x vs baseline; 1.0115x over previous
import jax
import jax.numpy as jnp
from jax import lax
from jax.experimental import pallas as pl
from jax.experimental.pallas import tpu as pltpu

N_DEV = 4
WINDOW = 128
NEG = -1e9
CR = 128
CHUNKS = ((0, 0, 0), (0, 128, 0), (1, 256, 128))
NCH = len(CHUNKS)

_DevId = getattr(pl, "DeviceIdType", None) or pltpu.DeviceIdType
_sem_signal = getattr(pl, "semaphore_signal", None) or pltpu.semaphore_signal
_sem_wait = getattr(pl, "semaphore_wait", None) or pltpu.semaphore_wait


def kernel(x, Wq, K_ext, V_ext, Wo):
    B, Sq, D = x.shape
    Dh = 64
    H = Wq.shape[1] // Dh
    HD = H * Dh
    SqQ = Sq // N_DEV
    SqH = Sq - CR

    K2 = K_ext.reshape(B, K_ext.shape[1], -1)
    V2 = V_ext.reshape(B, V_ext.shape[1], -1)

    def body(x_ref, wq_ref, k_ref, v_ref, wo_ref, out_ref,
             ksend, vsend, krecv, vrecv, pbuf, rsrecv, agsend, agrecv,
             ksend_sems, vsend_sems, krecv_sems, vrecv_sems,
             rssend_sems, rsrecv_sems, agsend_sems, agrecv_sems):
        my = lax.axis_index("i")

        bar = pltpu.get_barrier_semaphore()
        for k in range(1, N_DEV):
            _sem_signal(bar, inc=1, device_id=((my + k) % N_DEV,),
                        device_id_type=_DevId.MESH)
        _sem_wait(bar, N_DEV - 1)

        for ci, (owner, col0, _) in enumerate(CHUNKS):
            r0 = col0 - owner * 256
            @pl.when(my == owner)
            def _(ci=ci, owner=owner, r0=r0):
                dsts = [(owner + 2) % N_DEV, (owner + 1) % N_DEV,
                        (owner + 3) % N_DEV]
                for send, recv, ssems, rsems, ref in (
                        (ksend, krecv, ksend_sems, krecv_sems, k_ref),
                        (vsend, vrecv, vsend_sems, vrecv_sems, v_ref)):
                    for cnt, j in enumerate(dsts):
                        send[ci, cnt] = ref[:, r0:r0 + CR,
                                            j * HD:(j + 1) * HD].astype(jnp.bfloat16)
                        pltpu.make_async_remote_copy(
                            src_ref=send.at[ci, cnt], dst_ref=recv.at[ci],
                            send_sem=ssems.at[ci, cnt], recv_sem=rsems.at[ci],
                            device_id=(j,), device_id_type=_DevId.MESH,
                        ).start()
                krecv[ci] = k_ref[:, r0:r0 + CR,
                                  owner * HD:(owner + 1) * HD].astype(jnp.bfloat16)
                vrecv[ci] = v_ref[:, r0:r0 + CR,
                                  owner * HD:(owner + 1) * HD].astype(jnp.bfloat16)

        Q = []
        for b in range(B):
            Q.append((lax.dot_general(
                x_ref[b].astype(jnp.bfloat16), wq_ref[:, :].astype(jnp.bfloat16),
                (((1,), (0,)), ((), ())), preferred_element_type=jnp.float32)
                * (0.125 * 1.4426950408889634)).astype(jnp.bfloat16))

        def bias(nrow, row0, col0):
            r = lax.broadcasted_iota(jnp.int32, (nrow, CR), 0) + row0
            c = lax.broadcasted_iota(jnp.int32, (nrow, CR), 1) + col0
            return jnp.where(jnp.abs(r - c) <= WINDOW, 0.0, NEG).astype(jnp.float32)

        def wait_one(ci, owner, recv, ssems, rsems):
            @pl.when(my != owner)
            def _():
                pltpu.make_async_remote_copy(
                    src_ref=recv.at[ci], dst_ref=recv.at[ci],
                    send_sem=ssems.at[ci, 0], recv_sem=rsems.at[ci],
                    device_id=(0,), device_id_type=_DevId.MESH,
                ).wait_recv()

        es = [[[None] * H for _ in range(B)] for _ in CHUNKS]
        ls = [[[None] * H for _ in range(B)] for _ in CHUNKS]
        pv = [[[None] * H for _ in range(B)] for _ in CHUNKS]
        for ci, (owner, col0, qrow0) in enumerate(CHUNKS):
            nrow = Sq - qrow0
            bci = bias(nrow, qrow0, col0)
            wait_one(ci, owner, krecv, ksend_sems, krecv_sems)
            for b in range(B):
                for h in range(H):
                    q = Q[b][qrow0:, h * Dh:(h + 1) * Dh]
                    e = jnp.exp2(lax.dot_general(
                        q, krecv[ci, b, :, h * Dh:(h + 1) * Dh],
                        (((1,), (1,)), ((), ())),
                        preferred_element_type=jnp.float32) + bci)
                    ls[ci][b][h] = jnp.sum(e, axis=1, keepdims=True)
                    es[ci][b][h] = e.astype(jnp.bfloat16)
            wait_one(ci, owner, vrecv, vsend_sems, vrecv_sems)
            for b in range(B):
                for h in range(H):
                    pv[ci][b][h] = lax.dot_general(
                        es[ci][b][h], vrecv[ci, b, :, h * Dh:(h + 1) * Dh],
                        (((1,), (0,)), ((), ())),
                        preferred_element_type=jnp.float32)

        for b in range(B):
            acc = jnp.zeros((Sq, D), jnp.float32)
            for h in range(H):
                c01 = pv[0][b][h] + pv[1][b][h]
                l01 = ls[0][b][h] + ls[1][b][h]
                ctx_hi = c01[SqH:, :] + pv[2][b][h]
                ctx = jnp.concatenate(
                    [c01[:SqH, :] * (1.0 / l01[:SqH, :]),
                     ctx_hi * (1.0 / (l01[SqH:, :] + ls[2][b][h]))], axis=0)
                acc = acc + lax.dot_general(
                    ctx.astype(jnp.bfloat16),
                    wo_ref[h * Dh:(h + 1) * Dh, :].astype(jnp.bfloat16),
                    (((1,), (0,)), ((), ())), preferred_element_type=jnp.float32)
            pbuf[b] = acc.astype(jnp.bfloat16)
            for k in (2, 1, 3):
                d = (my + k) % N_DEV
                pltpu.make_async_remote_copy(
                    src_ref=pbuf.at[b, pl.ds(d * SqQ, SqQ), :],
                    dst_ref=rsrecv.at[N_DEV - 1 - k, b],
                    send_sem=rssend_sems.at[k - 1, b],
                    recv_sem=rsrecv_sems.at[N_DEV - 1 - k, b],
                    device_id=(d,), device_id_type=_DevId.MESH,
                ).start()

        for ci, (owner, col0, _) in enumerate(CHUNKS):
            @pl.when(my == owner)
            def _(ci=ci):
                for cnt in range(N_DEV - 1):
                    pltpu.make_async_remote_copy(
                        src_ref=ksend.at[ci, cnt], dst_ref=krecv.at[ci],
                        send_sem=ksend_sems.at[ci, cnt], recv_sem=krecv_sems.at[ci],
                        device_id=(0,), device_id_type=_DevId.MESH,
                    ).wait_send()
                    pltpu.make_async_remote_copy(
                        src_ref=vsend.at[ci, cnt], dst_ref=vrecv.at[ci],
                        send_sem=vsend_sems.at[ci, cnt], recv_sem=vrecv_sems.at[ci],
                        device_id=(0,), device_id_type=_DevId.MESH,
                    ).wait_send()

        sums = []
        for b in range(B):
            for j in range(N_DEV - 1):
                pltpu.make_async_remote_copy(
                    src_ref=rsrecv.at[j, b], dst_ref=rsrecv.at[j, b],
                    send_sem=rssend_sems.at[0, b], recv_sem=rsrecv_sems.at[j, b],
                    device_id=(0,), device_id_type=_DevId.MESH,
                ).wait_recv()
            sum_q = pbuf[b, pl.ds(my * SqQ, SqQ), :].astype(jnp.float32)
            for j in range(N_DEV - 1):
                sum_q = sum_q + rsrecv[j, b].astype(jnp.float32)
            sums.append(sum_q)
            agsend[b] = sum_q.astype(jnp.bfloat16)
            for k in (2, 1, 3):
                pltpu.make_async_remote_copy(
                    src_ref=agsend.at[b], dst_ref=agrecv.at[N_DEV - 1 - k, b],
                    send_sem=agsend_sems.at[k - 1, b],
                    recv_sem=agrecv_sems.at[N_DEV - 1 - k, b],
                    device_id=((my + k) % N_DEV,), device_id_type=_DevId.MESH,
                ).start()

        for b in range(B):
            out_ref[b, pl.ds(my * SqQ, SqQ), :] = sums[b]
            for j in range(N_DEV - 1):
                pltpu.make_async_remote_copy(
                    src_ref=agsend.at[b], dst_ref=agrecv.at[j, b],
                    send_sem=agsend_sems.at[0, b], recv_sem=agrecv_sems.at[j, b],
                    device_id=(0,), device_id_type=_DevId.MESH,
                ).wait_recv()
                src = (my + j + 1) % N_DEV
                out_ref[b, pl.ds(src * SqQ, SqQ), :] = \
                    agrecv[j, b].astype(jnp.float32)

        for b in range(B):
            for k in range(1, N_DEV):
                pltpu.make_async_remote_copy(
                    src_ref=pbuf.at[b, pl.ds(0, SqQ), :], dst_ref=rsrecv.at[0, b],
                    send_sem=rssend_sems.at[k - 1, b], recv_sem=rsrecv_sems.at[0, b],
                    device_id=(0,), device_id_type=_DevId.MESH,
                ).wait_send()
                pltpu.make_async_remote_copy(
                    src_ref=agsend.at[b], dst_ref=agrecv.at[0, b],
                    send_sem=agsend_sems.at[k - 1, b], recv_sem=agrecv_sems.at[0, b],
                    device_id=(0,), device_id_type=_DevId.MESH,
                ).wait_send()

    return pl.pallas_call(
        body,
        out_shape=jax.ShapeDtypeStruct((B, Sq, D), jnp.float32),
        in_specs=[pl.BlockSpec(memory_space=pltpu.VMEM)] * 5,
        out_specs=pl.BlockSpec(memory_space=pltpu.VMEM),
        scratch_shapes=[
            pltpu.VMEM((NCH, N_DEV - 1, B, CR, HD), jnp.bfloat16),
            pltpu.VMEM((NCH, N_DEV - 1, B, CR, HD), jnp.bfloat16),
            pltpu.VMEM((NCH, B, CR, HD), jnp.bfloat16),
            pltpu.VMEM((NCH, B, CR, HD), jnp.bfloat16),
            pltpu.VMEM((B, Sq, D), jnp.bfloat16),
            pltpu.VMEM((N_DEV - 1, B, Sq // N_DEV, D), jnp.bfloat16),
            pltpu.VMEM((B, Sq // N_DEV, D), jnp.bfloat16),
            pltpu.VMEM((N_DEV - 1, B, Sq // N_DEV, D), jnp.bfloat16),
            pltpu.SemaphoreType.DMA((NCH, N_DEV - 1)),
            pltpu.SemaphoreType.DMA((NCH, N_DEV - 1)),
            pltpu.SemaphoreType.DMA((NCH,)),
            pltpu.SemaphoreType.DMA((NCH,)),
            pltpu.SemaphoreType.DMA((N_DEV - 1, B)),
            pltpu.SemaphoreType.DMA((N_DEV - 1, B)),
            pltpu.SemaphoreType.DMA((N_DEV - 1, B)),
            pltpu.SemaphoreType.DMA((N_DEV - 1, B)),
        ],
        compiler_params=pltpu.CompilerParams(collective_id=0),
    )(x, Wq, K2, V2, Wo)
